# Initial kernel scaffold; baseline (speedup 1.0000x reference)
import functools

import numpy as np
import jax
import jax.numpy as jnp
from jax import lax
from jax.experimental import pallas as pl
from jax.experimental.pallas import tpu as pltpu
from jax.experimental.pallas import tpu_sc as plsc

N = 10000
E = 320000
D = 128
WR = 144
NC = 2
NS = 16
NW = NC * NS
EPW = E // NW
CHUNK = 80
NCHUNK = EPW // CHUNK
ROWS_PER_SUB = N // NS
BN = 1000

f32 = jnp.float32


def _np_consts():
    P = np.zeros((128, 128), np.float32)
    for k in range(8):
        for l in range(16):
            P[(l % 4) * 32 + k * 4 + l // 4, k * 16 + l] = 1.0
    SEGREP = np.zeros((128, 16), np.float32)
    for c in range(128):
        for j in range(16):
            if c // 32 == j % 4:
                SEGREP[c, j] = 1.0
    EXPAND = np.zeros((16, 128), np.float32)
    for c in range(128):
        EXPAND[c // 32, c] = 1.0
    REP1 = np.ones((128, 16), np.float32)
    EXP2 = np.zeros((16, 128), np.float32)
    EXP2[0, :] = 1.0
    return P, P.T.copy(), SEGREP, EXPAND, REP1, EXP2


_P, _PT, _SEGREP, _EXPAND, _REP1, _EXP2 = _np_consts()


def _tc_layer1(x, W1, asf, adf, P, SEGREP):
    def body(x_ref, w_ref, as_ref, ad_ref, p_ref, seg_ref, hrow_ref, adp_ref):
        h = jnp.dot(x_ref[...], w_ref[...], preferred_element_type=f32)
        asrep = jnp.dot(h * as_ref[...], seg_ref[...], preferred_element_type=f32)
        adrep = jnp.dot(h * ad_ref[...], seg_ref[...], preferred_element_type=f32)
        hperm = jnp.dot(h, p_ref[...], preferred_element_type=f32)
        hrow_ref[...] = jnp.concatenate([hperm, asrep], axis=1)
        adp_ref[...] = adrep

    return pl.pallas_call(
        body,
        grid=(N // BN,),
        in_specs=[
            pl.BlockSpec((BN, 128), lambda i: (i, 0)),
            pl.BlockSpec((128, 128), lambda i: (0, 0)),
            pl.BlockSpec((1, 128), lambda i: (0, 0)),
            pl.BlockSpec((1, 128), lambda i: (0, 0)),
            pl.BlockSpec((128, 128), lambda i: (0, 0)),
            pl.BlockSpec((128, 16), lambda i: (0, 0)),
        ],
        out_specs=[
            pl.BlockSpec((BN, WR), lambda i: (i, 0)),
            pl.BlockSpec((BN, 16), lambda i: (i, 0)),
        ],
        out_shape=[
            jax.ShapeDtypeStruct((N, WR), f32),
            jax.ShapeDtypeStruct((N, 16), f32),
        ],
    )(x, W1, asf, adf, P, SEGREP)


def _tc_mid(acc0, acc1, b1r, W2, asf2, adf2, PT, EXPAND, REP1):
    def body(a0, a1, b1_ref, w2_ref, as2_ref, ad2_ref, pt_ref, ex_ref,
             rep_ref, hrow2_ref, adp2_ref):
        acc = a0[...] + a1[...]
        numer = jnp.dot(acc[:, :128], pt_ref[...], preferred_element_type=f32)
        den = jnp.dot(acc[:, 128:144], ex_ref[...], preferred_element_type=f32)
        out1 = numer / (den + 1e-16) + b1_ref[...]
        g = jnp.where(out1 > 0, out1, jnp.expm1(out1))
        h2 = jnp.dot(g, w2_ref[...], preferred_element_type=f32)
        as2 = jnp.dot(h2 * as2_ref[...], rep_ref[...], preferred_element_type=f32)
        ad2 = jnp.dot(h2 * ad2_ref[...], rep_ref[...], preferred_element_type=f32)
        hrow2_ref[...] = jnp.concatenate([h2, as2], axis=1)
        adp2_ref[...] = ad2

    return pl.pallas_call(
        body,
        grid=(N // BN,),
        in_specs=[
            pl.BlockSpec((BN, WR), lambda i: (i, 0)),
            pl.BlockSpec((BN, WR), lambda i: (i, 0)),
            pl.BlockSpec((1, 128), lambda i: (0, 0)),
            pl.BlockSpec((128, 128), lambda i: (0, 0)),
            pl.BlockSpec((1, 128), lambda i: (0, 0)),
            pl.BlockSpec((1, 128), lambda i: (0, 0)),
            pl.BlockSpec((128, 128), lambda i: (0, 0)),
            pl.BlockSpec((16, 128), lambda i: (0, 0)),
            pl.BlockSpec((128, 16), lambda i: (0, 0)),
        ],
        out_specs=[
            pl.BlockSpec((BN, WR), lambda i: (i, 0)),
            pl.BlockSpec((BN, 16), lambda i: (i, 0)),
        ],
        out_shape=[
            jax.ShapeDtypeStruct((N, WR), f32),
            jax.ShapeDtypeStruct((N, 16), f32),
        ],
    )(acc0, acc1, b1r, W2, asf2, adf2, PT, EXPAND, REP1)


def _tc_final(acc0, acc1, b2r, EXP2):
    def body(a0, a1, b2_ref, ex2_ref, out_ref):
        acc = a0[...] + a1[...]
        den = jnp.dot(acc[:, 128:144], ex2_ref[...], preferred_element_type=f32)
        out_ref[...] = acc[:, :128] / (den + 1e-16) + b2_ref[...]

    return pl.pallas_call(
        body,
        grid=(N // BN,),
        in_specs=[
            pl.BlockSpec((BN, WR), lambda i: (i, 0)),
            pl.BlockSpec((BN, WR), lambda i: (i, 0)),
            pl.BlockSpec((1, 128), lambda i: (0, 0)),
            pl.BlockSpec((16, 128), lambda i: (0, 0)),
        ],
        out_specs=pl.BlockSpec((BN, 128), lambda i: (i, 0)),
        out_shape=jax.ShapeDtypeStruct((N, 128), f32),
    )(acc0, acc1, b2r, EXP2)


def _make_sc_agg():
    mesh = plsc.VectorSubcoreMesh(core_axis_name="c", subcore_axis_name="s")

    @functools.partial(
        pl.kernel,
        out_type=(jax.ShapeDtypeStruct((N, WR), f32),
                  jax.ShapeDtypeStruct((N, WR), f32)),
        mesh=mesh,
        scratch_types=[
            pltpu.VMEM((CHUNK, WR), f32),
            pltpu.VMEM((CHUNK, 16), f32),
            pltpu.VMEM((CHUNK,), jnp.int32),
            pltpu.VMEM((CHUNK,), jnp.int32),
            pltpu.VMEM_SHARED((N, WR), f32),
        ],
    )
    def k(hrow, adp, src, dst, zeros, acc0_out, acc1_out,
          g_buf, ad_buf, sidx, didx, acc):
        cid = lax.axis_index("c")
        sid = lax.axis_index("s")
        wid = sid * NC + cid
        zbase = sid * ROWS_PER_SUB
        pltpu.sync_copy(zeros, acc.at[pl.ds(zbase, ROWS_PER_SUB)])
        plsc.subcore_barrier()
        ebase = wid * EPW

        @pl.loop(0, NCHUNK)
        def _(ci):
            base = pl.multiple_of(ebase + ci * CHUNK, 8)
            pltpu.sync_copy(src.at[pl.ds(base, CHUNK)], sidx)
            pltpu.sync_copy(dst.at[pl.ds(base, CHUNK)], didx)
            pltpu.sync_copy(hrow.at[sidx], g_buf)
            pltpu.sync_copy(adp.at[didx], ad_buf)

            @pl.loop(0, CHUNK)
            def _(e):
                a = g_buf[e, pl.ds(128, 16)]
                b = ad_buf[e, pl.ds(0, 16)]
                s = a + b
                s = jnp.maximum(s, 0.2 * s)
                w = jnp.exp(s)
                for kk in range(8):
                    v = g_buf[e, pl.ds(kk * 16, 16)]
                    g_buf[e, pl.ds(kk * 16, 16)] = v * w
                g_buf[e, pl.ds(128, 16)] = w

            pltpu.sync_copy(g_buf, acc.at[didx], add=True)

        plsc.subcore_barrier()

        @pl.when(cid == 0)
        def _():
            pltpu.sync_copy(acc.at[pl.ds(zbase, ROWS_PER_SUB)],
                            acc0_out.at[pl.ds(zbase, ROWS_PER_SUB)])

        @pl.when(cid == 1)
        def _():
            pltpu.sync_copy(acc.at[pl.ds(zbase, ROWS_PER_SUB)],
                            acc1_out.at[pl.ds(zbase, ROWS_PER_SUB)])

    return k


_sc_agg = _make_sc_agg()


def kernel(x, edge_index, W1, a_src1, a_dst1, b1, W2, a_src2, a_dst2, b2):
    src = edge_index[0]
    dst = edge_index[1]
    asf1 = a_src1.reshape(1, 128)
    adf1 = a_dst1.reshape(1, 128)
    asf2 = a_src2.reshape(1, 128)
    adf2 = a_dst2.reshape(1, 128)
    b1r = b1.reshape(1, 128)
    b2r = b2.reshape(1, 128)
    zeros = jnp.zeros((ROWS_PER_SUB, WR), f32)
    P = jnp.asarray(_P)
    PT = jnp.asarray(_PT)
    SEGREP = jnp.asarray(_SEGREP)
    EXPAND = jnp.asarray(_EXPAND)
    REP1 = jnp.asarray(_REP1)
    EXP2 = jnp.asarray(_EXP2)

    hrow, adp = _tc_layer1(x, W1, asf1, adf1, P, SEGREP)
    acc0, acc1 = _sc_agg(hrow, adp, src, dst, zeros)
    hrow2, adp2 = _tc_mid(acc0, acc1, b1r, W2, asf2, adf2, PT, EXPAND, REP1)
    acc0b, acc1b = _sc_agg(hrow2, adp2, src, dst, zeros)
    out = _tc_final(acc0b, acc1b, b2r, EXP2)
    return out


# speedup vs baseline: 27.5464x; 27.5464x over previous
import functools

import numpy as np
import jax
import jax.numpy as jnp
from jax import lax
from jax.experimental import pallas as pl
from jax.experimental.pallas import tpu as pltpu
from jax.experimental.pallas import tpu_sc as plsc

N = 10000
E = 320000
D = 128
WR = 144
NC = 2
NS = 16
NW = NC * NS
EPW = E // NW
CHUNK = 80
NCHUNK = EPW // CHUNK
ROWS_PER_SUB = N // NS
BN = 1000

f32 = jnp.float32


def _np_consts():
    P = np.zeros((128, 128), np.float32)
    for k in range(8):
        for l in range(16):
            P[(l % 4) * 32 + k * 4 + l // 4, k * 16 + l] = 1.0
    SEGREP = np.zeros((128, 16), np.float32)
    for c in range(128):
        for j in range(16):
            if c // 32 == j % 4:
                SEGREP[c, j] = 1.0
    EXPAND = np.zeros((16, 128), np.float32)
    for c in range(128):
        EXPAND[c // 32, c] = 1.0
    REP1 = np.ones((128, 16), np.float32)
    EXP2 = np.zeros((16, 128), np.float32)
    EXP2[0, :] = 1.0
    return P, P.T.copy(), SEGREP, EXPAND, REP1, EXP2


_P, _PT, _SEGREP, _EXPAND, _REP1, _EXP2 = _np_consts()


def _tc_layer1(x, W1, asf, adf, P, SEGREP):
    def body(x_ref, w_ref, as_ref, ad_ref, p_ref, seg_ref, hrow_ref, adp_ref):
        h = jnp.dot(x_ref[...], w_ref[...], preferred_element_type=f32)
        asrep = jnp.dot(h * as_ref[...], seg_ref[...], preferred_element_type=f32)
        adrep = jnp.dot(h * ad_ref[...], seg_ref[...], preferred_element_type=f32)
        hperm = jnp.dot(h, p_ref[...], preferred_element_type=f32)
        hrow_ref[...] = jnp.concatenate([hperm, asrep], axis=1)
        adp_ref[...] = adrep

    return pl.pallas_call(
        body,
        grid=(N // BN,),
        in_specs=[
            pl.BlockSpec((BN, 128), lambda i: (i, 0)),
            pl.BlockSpec((128, 128), lambda i: (0, 0)),
            pl.BlockSpec((1, 128), lambda i: (0, 0)),
            pl.BlockSpec((1, 128), lambda i: (0, 0)),
            pl.BlockSpec((128, 128), lambda i: (0, 0)),
            pl.BlockSpec((128, 16), lambda i: (0, 0)),
        ],
        out_specs=[
            pl.BlockSpec((BN, WR), lambda i: (i, 0)),
            pl.BlockSpec((BN, 16), lambda i: (i, 0)),
        ],
        out_shape=[
            jax.ShapeDtypeStruct((N, WR), f32),
            jax.ShapeDtypeStruct((N, 16), f32),
        ],
    )(x, W1, asf, adf, P, SEGREP)


def _tc_mid(acc0, acc1, b1r, W2, asf2, adf2, PT, EXPAND, REP1):
    def body(a0, a1, b1_ref, w2_ref, as2_ref, ad2_ref, pt_ref, ex_ref,
             rep_ref, hrow2_ref, adp2_ref):
        acc = a0[...] + a1[...]
        numer = jnp.dot(acc[:, :128], pt_ref[...], preferred_element_type=f32)
        den = jnp.dot(acc[:, 128:144], ex_ref[...], preferred_element_type=f32)
        out1 = numer / (den + 1e-16) + b1_ref[...]
        g = jnp.where(out1 > 0, out1, jnp.exp(out1) - 1.0)
        h2 = jnp.dot(g, w2_ref[...], preferred_element_type=f32)
        as2 = jnp.dot(h2 * as2_ref[...], rep_ref[...], preferred_element_type=f32)
        ad2 = jnp.dot(h2 * ad2_ref[...], rep_ref[...], preferred_element_type=f32)
        hrow2_ref[...] = jnp.concatenate([h2, as2], axis=1)
        adp2_ref[...] = ad2

    return pl.pallas_call(
        body,
        grid=(N // BN,),
        in_specs=[
            pl.BlockSpec((BN, WR), lambda i: (i, 0)),
            pl.BlockSpec((BN, WR), lambda i: (i, 0)),
            pl.BlockSpec((1, 128), lambda i: (0, 0)),
            pl.BlockSpec((128, 128), lambda i: (0, 0)),
            pl.BlockSpec((1, 128), lambda i: (0, 0)),
            pl.BlockSpec((1, 128), lambda i: (0, 0)),
            pl.BlockSpec((128, 128), lambda i: (0, 0)),
            pl.BlockSpec((16, 128), lambda i: (0, 0)),
            pl.BlockSpec((128, 16), lambda i: (0, 0)),
        ],
        out_specs=[
            pl.BlockSpec((BN, WR), lambda i: (i, 0)),
            pl.BlockSpec((BN, 16), lambda i: (i, 0)),
        ],
        out_shape=[
            jax.ShapeDtypeStruct((N, WR), f32),
            jax.ShapeDtypeStruct((N, 16), f32),
        ],
    )(acc0, acc1, b1r, W2, asf2, adf2, PT, EXPAND, REP1)


def _tc_final(acc0, acc1, b2r, EXP2):
    def body(a0, a1, b2_ref, ex2_ref, out_ref):
        acc = a0[...] + a1[...]
        den = jnp.dot(acc[:, 128:144], ex2_ref[...], preferred_element_type=f32)
        out_ref[...] = acc[:, :128] / (den + 1e-16) + b2_ref[...]

    return pl.pallas_call(
        body,
        grid=(N // BN,),
        in_specs=[
            pl.BlockSpec((BN, WR), lambda i: (i, 0)),
            pl.BlockSpec((BN, WR), lambda i: (i, 0)),
            pl.BlockSpec((1, 128), lambda i: (0, 0)),
            pl.BlockSpec((16, 128), lambda i: (0, 0)),
        ],
        out_specs=pl.BlockSpec((BN, 128), lambda i: (i, 0)),
        out_shape=jax.ShapeDtypeStruct((N, 128), f32),
    )(acc0, acc1, b2r, EXP2)


def _make_sc_agg():
    mesh = plsc.VectorSubcoreMesh(core_axis_name="c", subcore_axis_name="s")

    @functools.partial(
        pl.kernel,
        out_type=(jax.ShapeDtypeStruct((N, WR), f32),
                  jax.ShapeDtypeStruct((N, WR), f32)),
        mesh=mesh,
        compiler_params=pltpu.CompilerParams(use_tc_tiling_on_sc=False),
        scratch_types=[
            pltpu.VMEM((CHUNK, WR), f32),
            pltpu.VMEM((CHUNK, 16), f32),
            pltpu.VMEM((CHUNK,), jnp.int32),
            pltpu.VMEM((CHUNK,), jnp.int32),
            pltpu.VMEM_SHARED((N, WR), f32),
        ],
    )
    def k(hrow, adp, src, dst, zeros, acc0_out, acc1_out,
          g_buf, ad_buf, sidx, didx, acc):
        cid = lax.axis_index("c")
        sid = lax.axis_index("s")
        wid = sid * NC + cid
        zbase = sid * ROWS_PER_SUB
        pltpu.sync_copy(zeros, acc.at[pl.ds(zbase, ROWS_PER_SUB)])
        plsc.subcore_barrier()
        ebase = wid * EPW

        @pl.loop(0, NCHUNK)
        def _(ci):
            base = pl.multiple_of(ebase + ci * CHUNK, 8)
            pltpu.sync_copy(src.at[pl.ds(base, CHUNK)], sidx)
            pltpu.sync_copy(dst.at[pl.ds(base, CHUNK)], didx)
            pltpu.sync_copy(hrow.at[sidx], g_buf)
            pltpu.sync_copy(adp.at[didx], ad_buf)

            @pl.loop(0, CHUNK)
            def _(e):
                a = g_buf[e, pl.ds(128, 16)]
                b = ad_buf[e, pl.ds(0, 16)]
                s = a + b
                s = jnp.maximum(s, 0.2 * s)
                w = jnp.exp(s)
                for kk in range(8):
                    v = g_buf[e, pl.ds(kk * 16, 16)]
                    g_buf[e, pl.ds(kk * 16, 16)] = v * w
                g_buf[e, pl.ds(128, 16)] = w

            pltpu.sync_copy(g_buf, acc.at[didx], add=True)

        plsc.subcore_barrier()

        @pl.when(cid == 0)
        def _():
            pltpu.sync_copy(acc.at[pl.ds(zbase, ROWS_PER_SUB)],
                            acc0_out.at[pl.ds(zbase, ROWS_PER_SUB)])

        @pl.when(cid == 1)
        def _():
            pltpu.sync_copy(acc.at[pl.ds(zbase, ROWS_PER_SUB)],
                            acc1_out.at[pl.ds(zbase, ROWS_PER_SUB)])

    return k


_sc_agg = _make_sc_agg()


def kernel(x, edge_index, W1, a_src1, a_dst1, b1, W2, a_src2, a_dst2, b2):
    src = edge_index[0]
    dst = edge_index[1]
    asf1 = a_src1.reshape(1, 128)
    adf1 = a_dst1.reshape(1, 128)
    asf2 = a_src2.reshape(1, 128)
    adf2 = a_dst2.reshape(1, 128)
    b1r = b1.reshape(1, 128)
    b2r = b2.reshape(1, 128)
    zeros = jnp.zeros((ROWS_PER_SUB, WR), f32)
    P = jnp.asarray(_P)
    PT = jnp.asarray(_PT)
    SEGREP = jnp.asarray(_SEGREP)
    EXPAND = jnp.asarray(_EXPAND)
    REP1 = jnp.asarray(_REP1)
    EXP2 = jnp.asarray(_EXP2)

    hrow, adp = _tc_layer1(x, W1, asf1, adf1, P, SEGREP)
    acc0, acc1 = _sc_agg(hrow, adp, src, dst, zeros)
    hrow2, adp2 = _tc_mid(acc0, acc1, b1r, W2, asf2, adf2, PT, EXPAND, REP1)
    acc0b, acc1b = _sc_agg(hrow2, adp2, src, dst, zeros)
    out = _tc_final(acc0b, acc1b, b2r, EXP2)
    return out


# speedup vs baseline: 53.9930x; 1.9601x over previous
import functools

import numpy as np
import jax
import jax.numpy as jnp
from jax import lax
from jax.experimental import pallas as pl
from jax.experimental.pallas import tpu as pltpu
from jax.experimental.pallas import tpu_sc as plsc

N = 10000
E = 320000
D = 128
WR = 144
NC = 2
NS = 16
NW = NC * NS
EPW = E // NW
CHUNK = 80
NCHUNK = EPW // CHUNK
ROWS_PER_SUB = N // NS
BN = 1000

f32 = jnp.float32


def _np_consts():
    P = np.zeros((128, 128), np.float32)
    for k in range(8):
        for l in range(16):
            P[(l % 4) * 32 + k * 4 + l // 4, k * 16 + l] = 1.0
    SEGREP = np.zeros((128, 16), np.float32)
    for c in range(128):
        for j in range(16):
            if c // 32 == j % 4:
                SEGREP[c, j] = 1.0
    EXPAND = np.zeros((16, 128), np.float32)
    for c in range(128):
        EXPAND[c // 32, c] = 1.0
    REP1 = np.ones((128, 16), np.float32)
    EXP2 = np.zeros((16, 128), np.float32)
    EXP2[0, :] = 1.0
    return P, P.T.copy(), SEGREP, EXPAND, REP1, EXP2


_P, _PT, _SEGREP, _EXPAND, _REP1, _EXP2 = _np_consts()


def _tc_layer1(x, W1, asf, adf, P, SEGREP):
    def body(x_ref, w_ref, as_ref, ad_ref, p_ref, seg_ref, hrow_ref, adp_ref):
        h = jnp.dot(x_ref[...], w_ref[...], preferred_element_type=f32)
        asrep = jnp.dot(h * as_ref[...], seg_ref[...], preferred_element_type=f32)
        adrep = jnp.dot(h * ad_ref[...], seg_ref[...], preferred_element_type=f32)
        hperm = jnp.dot(h, p_ref[...], preferred_element_type=f32)
        hrow_ref[...] = jnp.concatenate([hperm, asrep], axis=1)
        adp_ref[...] = adrep

    return pl.pallas_call(
        body,
        grid=(N // BN,),
        in_specs=[
            pl.BlockSpec((BN, 128), lambda i: (i, 0)),
            pl.BlockSpec((128, 128), lambda i: (0, 0)),
            pl.BlockSpec((1, 128), lambda i: (0, 0)),
            pl.BlockSpec((1, 128), lambda i: (0, 0)),
            pl.BlockSpec((128, 128), lambda i: (0, 0)),
            pl.BlockSpec((128, 16), lambda i: (0, 0)),
        ],
        out_specs=[
            pl.BlockSpec((BN, WR), lambda i: (i, 0)),
            pl.BlockSpec((BN, 16), lambda i: (i, 0)),
        ],
        out_shape=[
            jax.ShapeDtypeStruct((N, WR), f32),
            jax.ShapeDtypeStruct((N, 16), f32),
        ],
    )(x, W1, asf, adf, P, SEGREP)


def _tc_mid(acc0, acc1, b1r, W2, asf2, adf2, PT, EXPAND, REP1):
    def body(a0, a1, b1_ref, w2_ref, as2_ref, ad2_ref, pt_ref, ex_ref,
             rep_ref, hrow2_ref, adp2_ref):
        acc = a0[...] + a1[...]
        numer = jnp.dot(acc[:, :128], pt_ref[...], preferred_element_type=f32)
        den = jnp.dot(acc[:, 128:144], ex_ref[...], preferred_element_type=f32)
        out1 = numer / (den + 1e-16) + b1_ref[...]
        g = jnp.where(out1 > 0, out1, jnp.exp(out1) - 1.0)
        h2 = jnp.dot(g, w2_ref[...], preferred_element_type=f32)
        as2 = jnp.dot(h2 * as2_ref[...], rep_ref[...], preferred_element_type=f32)
        ad2 = jnp.dot(h2 * ad2_ref[...], rep_ref[...], preferred_element_type=f32)
        hrow2_ref[...] = jnp.concatenate([h2, as2], axis=1)
        adp2_ref[...] = ad2

    return pl.pallas_call(
        body,
        grid=(N // BN,),
        in_specs=[
            pl.BlockSpec((BN, WR), lambda i: (i, 0)),
            pl.BlockSpec((BN, WR), lambda i: (i, 0)),
            pl.BlockSpec((1, 128), lambda i: (0, 0)),
            pl.BlockSpec((128, 128), lambda i: (0, 0)),
            pl.BlockSpec((1, 128), lambda i: (0, 0)),
            pl.BlockSpec((1, 128), lambda i: (0, 0)),
            pl.BlockSpec((128, 128), lambda i: (0, 0)),
            pl.BlockSpec((16, 128), lambda i: (0, 0)),
            pl.BlockSpec((128, 16), lambda i: (0, 0)),
        ],
        out_specs=[
            pl.BlockSpec((BN, WR), lambda i: (i, 0)),
            pl.BlockSpec((BN, 16), lambda i: (i, 0)),
        ],
        out_shape=[
            jax.ShapeDtypeStruct((N, WR), f32),
            jax.ShapeDtypeStruct((N, 16), f32),
        ],
    )(acc0, acc1, b1r, W2, asf2, adf2, PT, EXPAND, REP1)


def _tc_final(acc0, acc1, b2r, EXP2):
    def body(a0, a1, b2_ref, ex2_ref, out_ref):
        acc = a0[...] + a1[...]
        den = jnp.dot(acc[:, 128:144], ex2_ref[...], preferred_element_type=f32)
        out_ref[...] = acc[:, :128] / (den + 1e-16) + b2_ref[...]

    return pl.pallas_call(
        body,
        grid=(N // BN,),
        in_specs=[
            pl.BlockSpec((BN, WR), lambda i: (i, 0)),
            pl.BlockSpec((BN, WR), lambda i: (i, 0)),
            pl.BlockSpec((1, 128), lambda i: (0, 0)),
            pl.BlockSpec((16, 128), lambda i: (0, 0)),
        ],
        out_specs=pl.BlockSpec((BN, 128), lambda i: (i, 0)),
        out_shape=jax.ShapeDtypeStruct((N, 128), f32),
    )(acc0, acc1, b2r, EXP2)


def _make_sc_agg():
    mesh = plsc.VectorSubcoreMesh(core_axis_name="c", subcore_axis_name="s")

    @functools.partial(
        pl.kernel,
        out_type=(jax.ShapeDtypeStruct((N, WR), f32),
                  jax.ShapeDtypeStruct((N, WR), f32)),
        mesh=mesh,
        compiler_params=pltpu.CompilerParams(use_tc_tiling_on_sc=False),
        scratch_types=[
            pltpu.VMEM((CHUNK, WR), f32),
            pltpu.VMEM((CHUNK, WR), f32),
            pltpu.VMEM((CHUNK, 16), f32),
            pltpu.VMEM((CHUNK, 16), f32),
            pltpu.VMEM((NCHUNK, CHUNK), jnp.int32),
            pltpu.VMEM((CHUNK,), jnp.int32),
            pltpu.VMEM((CHUNK,), jnp.int32),
            pltpu.VMEM((CHUNK,), jnp.int32),
            pltpu.VMEM((CHUNK,), jnp.int32),
            pltpu.VMEM_SHARED((N, WR), f32),
            pltpu.SemaphoreType.DMA,
            pltpu.SemaphoreType.DMA,
        ],
    )
    def k(hrow, adp, pidx3, zeros, acc0_out, acc1_out,
          g_bufA, g_bufB, ad_bufA, ad_bufB, pidx_all,
          sidxA, sidxB, didxA, didxB, acc, semA, semB):
        cid = lax.axis_index("c")
        sid = lax.axis_index("s")
        wid = sid * NC + cid
        zbase = sid * ROWS_PER_SUB
        pltpu.sync_copy(pidx3.at[wid], pidx_all)
        pltpu.sync_copy(zeros, acc.at[pl.ds(zbase, ROWS_PER_SUB)])
        plsc.subcore_barrier()

        def fetch(c, g_buf, ad_buf, sidx, didx, sem):
            for g in range(CHUNK // 16):
                p = pidx_all[c, pl.ds(g * 16, 16)]
                sidx[pl.ds(g * 16, 16)] = p >> 14
                didx[pl.ds(g * 16, 16)] = p & 16383
            pltpu.async_copy(hrow.at[sidx], g_buf, sem)
            pltpu.async_copy(adp.at[didx], ad_buf, sem)

        def process(c, g_buf, ad_buf, sidx, didx, sem):
            pltpu.make_async_copy(hrow.at[sidx], g_buf, sem).wait()
            pltpu.make_async_copy(adp.at[didx], ad_buf, sem).wait()

            @pl.loop(0, CHUNK, unroll=2)
            def _(e):
                a = g_buf[e, pl.ds(128, 16)]
                b = ad_buf[e, pl.ds(0, 16)]
                s = a + b
                s = jnp.maximum(s, 0.2 * s)
                w = jnp.exp(s)
                for kk in range(8):
                    v = g_buf[e, pl.ds(kk * 16, 16)]
                    g_buf[e, pl.ds(kk * 16, 16)] = v * w
                g_buf[e, pl.ds(128, 16)] = w

            pltpu.sync_copy(g_buf, acc.at[didx], add=True)

        fetch(0, g_bufA, ad_bufA, sidxA, didxA, semA)

        @pl.loop(0, (NCHUNK - 1) // 2)
        def _(j):
            c = 2 * j
            fetch(c + 1, g_bufB, ad_bufB, sidxB, didxB, semB)
            process(c, g_bufA, ad_bufA, sidxA, didxA, semA)
            fetch(c + 2, g_bufA, ad_bufA, sidxA, didxA, semA)
            process(c + 1, g_bufB, ad_bufB, sidxB, didxB, semB)

        process(NCHUNK - 1, g_bufA, ad_bufA, sidxA, didxA, semA)

        plsc.subcore_barrier()

        @pl.when(cid == 0)
        def _():
            pltpu.sync_copy(acc.at[pl.ds(zbase, ROWS_PER_SUB)],
                            acc0_out.at[pl.ds(zbase, ROWS_PER_SUB)])

        @pl.when(cid == 1)
        def _():
            pltpu.sync_copy(acc.at[pl.ds(zbase, ROWS_PER_SUB)],
                            acc1_out.at[pl.ds(zbase, ROWS_PER_SUB)])

    return k


_sc_agg = _make_sc_agg()


def kernel(x, edge_index, W1, a_src1, a_dst1, b1, W2, a_src2, a_dst2, b2):
    pidx3 = (edge_index[0] * 16384 + edge_index[1]).reshape(NW, NCHUNK, CHUNK)
    asf1 = a_src1.reshape(1, 128)
    adf1 = a_dst1.reshape(1, 128)
    asf2 = a_src2.reshape(1, 128)
    adf2 = a_dst2.reshape(1, 128)
    b1r = b1.reshape(1, 128)
    b2r = b2.reshape(1, 128)
    zeros = jnp.zeros((ROWS_PER_SUB, WR), f32)
    P = jnp.asarray(_P)
    PT = jnp.asarray(_PT)
    SEGREP = jnp.asarray(_SEGREP)
    EXPAND = jnp.asarray(_EXPAND)
    REP1 = jnp.asarray(_REP1)
    EXP2 = jnp.asarray(_EXP2)

    hrow, adp = _tc_layer1(x, W1, asf1, adf1, P, SEGREP)
    acc0, acc1 = _sc_agg(hrow, adp, pidx3, zeros)
    hrow2, adp2 = _tc_mid(acc0, acc1, b1r, W2, asf2, adf2, PT, EXPAND, REP1)
    acc0b, acc1b = _sc_agg(hrow2, adp2, pidx3, zeros)
    out = _tc_final(acc0b, acc1b, b2r, EXP2)
    return out
